# baseline (speedup 1.0000x reference)
import jax
import jax.numpy as jnp
from jax.experimental import pallas as pl
from jax.experimental.pallas import tpu as pltpu

_T = 16
_BLK = 512


_NCH = 4


def _body(x_hbm, out_hbm, xv, ob0, ob1, in0, in1, in2, in3, s0, s1):
    B, F = xv.shape
    nblk = B // _BLK
    rows_c = B // _NCH

    insems = (in0, in1, in2, in3)
    cps = [
        pltpu.make_async_copy(
            x_hbm.at[pl.ds(c * rows_c, rows_c)],
            xv.at[pl.ds(c * rows_c, rows_c)],
            insems[c],
        )
        for c in range(_NCH)
    ]
    for cp in cps:
        cp.start()
    mn = mx = None
    for c in range(_NCH):
        cps[c].wait()
        blk = xv[pl.ds(c * rows_c, rows_c), :]
        bmin = jnp.min(blk)
        bmax = jnp.max(blk)
        mn = bmin if c == 0 else jnp.minimum(mn, bmin)
        mx = bmax if c == 0 else jnp.maximum(mx, bmax)
    scale = mx - mn + 1e-8

    bufs = (ob0, ob1)
    sems = (s0, s1)
    for i in range(nblk):
        buf = bufs[i % 2]
        sem = sems[i % 2]
        if i >= 2:
            pltpu.make_async_copy(
                buf, out_hbm.at[pl.ds((i - 2) * _BLK, _BLK)], sem
            ).wait()
        xblk = xv[pl.ds(i * _BLK, _BLK), :]
        xn = jnp.clip((xblk - mn) / scale, 0.0, 1.0)
        lat = ((1.0 - xn) * (_T - 1)).astype(jnp.int32)
        t = jax.lax.broadcasted_iota(jnp.int32, (_BLK, _T, F), 1)
        buf[...] = (lat[:, None, :] == t).astype(jnp.float32)
        pltpu.make_async_copy(
            buf, out_hbm.at[pl.ds(i * _BLK, _BLK)], sem
        ).start()
    for i in range(nblk - 2, nblk):
        pltpu.make_async_copy(
            bufs[i % 2], out_hbm.at[pl.ds(i * _BLK, _BLK)], sems[i % 2]
        ).wait()


def kernel(x):
    B, F = x.shape
    return pl.pallas_call(
        _body,
        in_specs=(pl.BlockSpec(memory_space=pl.ANY),),
        out_specs=pl.BlockSpec(memory_space=pl.ANY),
        out_shape=jax.ShapeDtypeStruct((B, _T, F), jnp.float32),
        scratch_shapes=[
            pltpu.VMEM((B, F), jnp.float32),
            pltpu.VMEM((_BLK, _T, F), jnp.float32),
            pltpu.VMEM((_BLK, _T, F), jnp.float32),
            pltpu.SemaphoreType.DMA,
            pltpu.SemaphoreType.DMA,
            pltpu.SemaphoreType.DMA,
            pltpu.SemaphoreType.DMA,
            pltpu.SemaphoreType.DMA,
            pltpu.SemaphoreType.DMA,
        ],
    )(x)


# speedup vs baseline: 1.0544x; 1.0544x over previous
import jax
import jax.numpy as jnp
from jax.experimental import pallas as pl
from jax.experimental.pallas import tpu as pltpu

_T = 16
_BLK = 256


_NCH = 4


def _body(x_hbm, out_hbm, xv, ob0, ob1, in0, in1, in2, in3, s0, s1):
    B, F = xv.shape
    nblk = B // _BLK
    rows_c = B // _NCH

    insems = (in0, in1, in2, in3)
    cps = [
        pltpu.make_async_copy(
            x_hbm.at[pl.ds(c * rows_c, rows_c)],
            xv.at[pl.ds(c * rows_c, rows_c)],
            insems[c],
        )
        for c in range(_NCH)
    ]
    for cp in cps:
        cp.start()
    mn = mx = None
    for c in range(_NCH):
        cps[c].wait()
        blk = xv[pl.ds(c * rows_c, rows_c), :]
        bmin = jnp.min(blk)
        bmax = jnp.max(blk)
        mn = bmin if c == 0 else jnp.minimum(mn, bmin)
        mx = bmax if c == 0 else jnp.maximum(mx, bmax)
    scale = mx - mn + 1e-8

    bufs = (ob0, ob1)
    sems = (s0, s1)
    for i in range(nblk):
        buf = bufs[i % 2]
        sem = sems[i % 2]
        if i >= 2:
            pltpu.make_async_copy(
                buf, out_hbm.at[pl.ds((i - 2) * _BLK, _BLK)], sem
            ).wait()
        xblk = xv[pl.ds(i * _BLK, _BLK), :]
        xn = jnp.clip((xblk - mn) / scale, 0.0, 1.0)
        lat = ((1.0 - xn) * (_T - 1)).astype(jnp.int32)
        t = jax.lax.broadcasted_iota(jnp.int32, (_BLK, _T, F), 1)
        buf[...] = (lat[:, None, :] == t).astype(jnp.float32)
        pltpu.make_async_copy(
            buf, out_hbm.at[pl.ds(i * _BLK, _BLK)], sem
        ).start()
    for i in range(nblk - 2, nblk):
        pltpu.make_async_copy(
            bufs[i % 2], out_hbm.at[pl.ds(i * _BLK, _BLK)], sems[i % 2]
        ).wait()


def kernel(x):
    B, F = x.shape
    return pl.pallas_call(
        _body,
        in_specs=(pl.BlockSpec(memory_space=pl.ANY),),
        out_specs=pl.BlockSpec(memory_space=pl.ANY),
        out_shape=jax.ShapeDtypeStruct((B, _T, F), jnp.float32),
        scratch_shapes=[
            pltpu.VMEM((B, F), jnp.float32),
            pltpu.VMEM((_BLK, _T, F), jnp.float32),
            pltpu.VMEM((_BLK, _T, F), jnp.float32),
            pltpu.SemaphoreType.DMA,
            pltpu.SemaphoreType.DMA,
            pltpu.SemaphoreType.DMA,
            pltpu.SemaphoreType.DMA,
            pltpu.SemaphoreType.DMA,
            pltpu.SemaphoreType.DMA,
        ],
    )(x)
